# baseline (speedup 1.0000x reference)
import functools

import jax
import jax.numpy as jnp
from jax import lax
from jax.experimental import pallas as pl
from jax.experimental.pallas import tpu as pltpu
from jax.experimental.pallas import tpu_sc as plsc

N_NODES = 10000
N_EDGES = 320000
D = 128
NC = 2
NS = 16
NW = NC * NS
CHUNK = 104
NCH = 98
E_PER_TILE = NCH * CHUNK
E_PAD = NW * E_PER_TILE
N_SRC = N_NODES + 8
ROWS_PER_TILE = 632
LAST_ROWS = N_NODES - 15 * ROWS_PER_TILE


def _sc_aggregate(x_pad, src, dst):
    mesh = plsc.VectorSubcoreMesh(core_axis_name="c", subcore_axis_name="s")

    @functools.partial(
        pl.kernel,
        mesh=mesh,
        out_type=jax.ShapeDtypeStruct((NC, N_NODES, D), jnp.float32),
        scratch_types=[
            pltpu.VMEM((E_PER_TILE,), jnp.int32),
            pltpu.VMEM((NCH, CHUNK), jnp.int32),
            pltpu.VMEM((CHUNK, D), jnp.float32),
            pltpu.VMEM((CHUNK, D), jnp.float32),
            pltpu.VMEM_SHARED((N_NODES, D), jnp.float32),
            pltpu.SemaphoreType.DMA,
            pltpu.SemaphoreType.DMA,
        ],
    )
    def k(x_hbm, src_hbm, dst_hbm, out_hbm, src_v, dst_v, rows0, rows1, acc,
          sem0, sem1):
        c = lax.axis_index("c")
        s = lax.axis_index("s")
        wid = s * NC + c

        pltpu.sync_copy(src_hbm.at[wid], src_v)
        pltpu.sync_copy(dst_hbm.at[wid], dst_v)

        @pl.when(s == 0)
        def _():
            pltpu.sync_copy(x_hbm.at[pl.ds(0, N_NODES)], acc)

        plsc.subcore_barrier()

        def gather_start(j, rows, sem):
            pltpu.async_copy(
                x_hbm.at[src_v.at[pl.ds(j * CHUNK, CHUNK)]], rows, sem)

        def gather_wait(j, rows, sem):
            pltpu.make_async_copy(
                x_hbm.at[src_v.at[pl.ds(j * CHUNK, CHUNK)]], rows, sem).wait()

        gather_start(0, rows0, sem0)
        gather_start(1, rows1, sem1)

        def body(i, carry):
            j0 = 2 * i
            gather_wait(j0, rows0, sem0)
            pltpu.sync_copy(rows0, acc.at[dst_v.at[j0]], add=True)

            @pl.when(i < NCH // 2 - 1)
            def _():
                gather_start(j0 + 2, rows0, sem0)

            gather_wait(j0 + 1, rows1, sem1)
            pltpu.sync_copy(rows1, acc.at[dst_v.at[j0 + 1]], add=True)

            @pl.when(i < NCH // 2 - 1)
            def _():
                gather_start(j0 + 3, rows1, sem1)

            return carry

        lax.fori_loop(0, NCH // 2, body, 0)

        plsc.subcore_barrier()

        @pl.when(s < NS - 1)
        def _():
            pltpu.sync_copy(
                acc.at[pl.ds(s * ROWS_PER_TILE, ROWS_PER_TILE)],
                out_hbm.at[c, pl.ds(s * ROWS_PER_TILE, ROWS_PER_TILE)],
            )

        @pl.when(s == NS - 1)
        def _():
            pltpu.sync_copy(
                acc.at[pl.ds(15 * ROWS_PER_TILE, LAST_ROWS)],
                out_hbm.at[c, pl.ds(15 * ROWS_PER_TILE, LAST_ROWS)],
            )

    return k(x_pad, src, dst)


def _mlp_body(p_ref, x_ref, w1_ref, b1_ref, g_ref, be_ref, w2_ref, b2_ref,
              o_ref):
    h = p_ref[0] + p_ref[1] - x_ref[...]
    h1 = jnp.dot(h, w1_ref[...], preferred_element_type=jnp.float32)
    h1 = h1 + b1_ref[...]
    mean = jnp.mean(h1, axis=0, keepdims=True)
    var = jnp.mean((h1 - mean) * (h1 - mean), axis=0, keepdims=True)
    hn = g_ref[...] * (h1 - mean) * lax.rsqrt(var + 1e-5) + be_ref[...]
    hn = jnp.maximum(hn, 0.0)
    o_ref[...] = (
        jnp.dot(hn, w2_ref[...], preferred_element_type=jnp.float32)
        + b2_ref[...]
    )


def _mlp(p, x, W1, b1, gamma, beta, W2, b2):
    return pl.pallas_call(
        _mlp_body,
        out_shape=jax.ShapeDtypeStruct((N_NODES, D), jnp.float32),
    )(p, x, W1, b1.reshape(1, D), gamma.reshape(1, D), beta.reshape(1, D),
      W2, b2.reshape(1, D))


def kernel(x, edge_index, edge_attr, W1, b1, gamma, beta, W2, b2):
    src = edge_index[0].astype(jnp.int32)
    dst = edge_index[1].astype(jnp.int32)
    pad = E_PAD - N_EDGES
    src_p = jnp.concatenate(
        [src, jnp.full((pad,), N_NODES, jnp.int32)]).reshape(NW, E_PER_TILE)
    dst_p = jnp.concatenate(
        [dst, jnp.zeros((pad,), jnp.int32)]).reshape(NW, NCH, CHUNK)
    x_pad = jnp.concatenate(
        [x, jnp.zeros((N_SRC - N_NODES, D), jnp.float32)], axis=0)
    partials = _sc_aggregate(x_pad, src_p, dst_p)
    return _mlp(partials, x, W1, b1, gamma, beta, W2, b2)


# speedup vs baseline: 1.0630x; 1.0630x over previous
import functools

import jax
import jax.numpy as jnp
from jax import lax
from jax.experimental import pallas as pl
from jax.experimental.pallas import tpu as pltpu
from jax.experimental.pallas import tpu_sc as plsc

N_NODES = 10000
N_EDGES = 320000
D = 128
NC = 2
NS = 16
NW = NC * NS
CHUNK = 96
HALF = CHUNK // 2
NCH = 106
E_PER_TILE = NCH * CHUNK
E_PAD = NW * E_PER_TILE
N_SRC = N_NODES + 8
ROWS_PER_TILE = 632
LAST_ROWS = N_NODES - 15 * ROWS_PER_TILE


def _sc_aggregate(x_pad, src, dst):
    mesh = plsc.VectorSubcoreMesh(core_axis_name="c", subcore_axis_name="s")

    @functools.partial(
        pl.kernel,
        mesh=mesh,
        out_type=jax.ShapeDtypeStruct((NC, N_NODES, D), jnp.float32),
        scratch_types=[
            pltpu.VMEM((E_PER_TILE,), jnp.int32),
            pltpu.VMEM((NCH, CHUNK), jnp.int32),
            pltpu.VMEM((CHUNK, D), jnp.float32),
            pltpu.VMEM((CHUNK, D), jnp.float32),
            pltpu.VMEM_SHARED((N_NODES, D), jnp.float32),
            pltpu.SemaphoreType.DMA,
            pltpu.SemaphoreType.DMA,
            pltpu.SemaphoreType.DMA,
            pltpu.SemaphoreType.DMA,
        ],
    )
    def k(x_hbm, src_hbm, dst_hbm, out_hbm, src_v, dst_v, rows0, rows1, acc,
          sem0a, sem0b, sem1a, sem1b):
        c = lax.axis_index("c")
        s = lax.axis_index("s")
        wid = s * NC + c

        pltpu.sync_copy(src_hbm.at[wid], src_v)
        pltpu.sync_copy(dst_hbm.at[wid], dst_v)

        @pl.when(s == 0)
        def _():
            pltpu.sync_copy(x_hbm.at[pl.ds(0, N_NODES)], acc)

        plsc.subcore_barrier()

        def gather_start(j, rows, sa, sb):
            pltpu.async_copy(
                x_hbm.at[src_v.at[pl.ds(j * CHUNK, HALF)]],
                rows.at[pl.ds(0, HALF)], sa)
            pltpu.async_copy(
                x_hbm.at[src_v.at[pl.ds(j * CHUNK + HALF, HALF)]],
                rows.at[pl.ds(HALF, HALF)], sb)

        def gather_wait(j, rows, sa, sb):
            pltpu.make_async_copy(
                x_hbm.at[src_v.at[pl.ds(j * CHUNK, HALF)]],
                rows.at[pl.ds(0, HALF)], sa).wait()
            pltpu.make_async_copy(
                x_hbm.at[src_v.at[pl.ds(j * CHUNK + HALF, HALF)]],
                rows.at[pl.ds(HALF, HALF)], sb).wait()

        gather_start(0, rows0, sem0a, sem0b)
        gather_start(1, rows1, sem1a, sem1b)

        def body(i, carry):
            j0 = 2 * i
            gather_wait(j0, rows0, sem0a, sem0b)
            pltpu.sync_copy(rows0, acc.at[dst_v.at[j0]], add=True)

            @pl.when(i < NCH // 2 - 1)
            def _():
                gather_start(j0 + 2, rows0, sem0a, sem0b)

            gather_wait(j0 + 1, rows1, sem1a, sem1b)
            pltpu.sync_copy(rows1, acc.at[dst_v.at[j0 + 1]], add=True)

            @pl.when(i < NCH // 2 - 1)
            def _():
                gather_start(j0 + 3, rows1, sem1a, sem1b)

            return carry

        lax.fori_loop(0, NCH // 2, body, 0)

        plsc.subcore_barrier()

        @pl.when(s < NS - 1)
        def _():
            pltpu.sync_copy(
                acc.at[pl.ds(s * ROWS_PER_TILE, ROWS_PER_TILE)],
                out_hbm.at[c, pl.ds(s * ROWS_PER_TILE, ROWS_PER_TILE)],
            )

        @pl.when(s == NS - 1)
        def _():
            pltpu.sync_copy(
                acc.at[pl.ds(15 * ROWS_PER_TILE, LAST_ROWS)],
                out_hbm.at[c, pl.ds(15 * ROWS_PER_TILE, LAST_ROWS)],
            )

    return k(x_pad, src, dst)


def _mlp_body(p_ref, x_ref, w1_ref, b1_ref, g_ref, be_ref, w2_ref, b2_ref,
              o_ref):
    h = p_ref[0] + p_ref[1] - x_ref[...]
    h1 = jnp.dot(h, w1_ref[...], preferred_element_type=jnp.float32)
    h1 = h1 + b1_ref[...]
    mean = jnp.mean(h1, axis=0, keepdims=True)
    var = jnp.mean((h1 - mean) * (h1 - mean), axis=0, keepdims=True)
    hn = g_ref[...] * (h1 - mean) * lax.rsqrt(var + 1e-5) + be_ref[...]
    hn = jnp.maximum(hn, 0.0)
    o_ref[...] = (
        jnp.dot(hn, w2_ref[...], preferred_element_type=jnp.float32)
        + b2_ref[...]
    )


def _mlp(p, x, W1, b1, gamma, beta, W2, b2):
    return pl.pallas_call(
        _mlp_body,
        out_shape=jax.ShapeDtypeStruct((N_NODES, D), jnp.float32),
    )(p, x, W1, b1.reshape(1, D), gamma.reshape(1, D), beta.reshape(1, D),
      W2, b2.reshape(1, D))


def kernel(x, edge_index, edge_attr, W1, b1, gamma, beta, W2, b2):
    src = edge_index[0].astype(jnp.int32)
    dst = edge_index[1].astype(jnp.int32)
    pad = E_PAD - N_EDGES
    src_p = jnp.concatenate(
        [src, jnp.full((pad,), N_NODES, jnp.int32)]).reshape(NW, E_PER_TILE)
    dst_p = jnp.concatenate(
        [dst, jnp.zeros((pad,), jnp.int32)]).reshape(NW, NCH, CHUNK)
    x_pad = jnp.concatenate(
        [x, jnp.zeros((N_SRC - N_NODES, D), jnp.float32)], axis=0)
    partials = _sc_aggregate(x_pad, src_p, dst_p)
    return _mlp(partials, x, W1, b1, gamma, beta, W2, b2)
